# Initial kernel scaffold; baseline (speedup 1.0000x reference)
import jax
import jax.numpy as jnp
from jax.experimental import pallas as pl


def kernel(logits, k, p, gumbel):
    raise NotImplementedError("write your pallas kernel here")


# speedup vs baseline: 45.0126x; 45.0126x over previous
import jax
import jax.numpy as jnp
from jax.experimental import pallas as pl
from jax.experimental.pallas import tpu as pltpu

_NEG_BIG = jnp.float32(-3.0e38)
_ROWS = 8


def _sortable(x):
    u = jax.lax.bitcast_convert_type(x, jnp.uint32)
    top = jnp.uint32(0x80000000)
    return jnp.where(u >= top, ~u, u | top)


def _block(k_ref, p_ref, x_ref, g_ref, out_ref, samp_ref, e_ref, s_ref):
    x = x_ref[:, :]
    kk = k_ref[:, :]
    pp = p_ref[:, :]

    m = jnp.max(x, axis=1, keepdims=True)
    s_ref[:, :] = _sortable(x)
    e_ref[:, :] = jnp.exp(x - m)

    lo0 = jnp.zeros(kk.shape, jnp.uint32)
    hi0 = jnp.full(kk.shape, jnp.uint32(0xFFFFFFFF))

    def cnt_step(_, carry):
        lo, hi = carry
        mid = lo + ((hi - lo) >> jnp.uint32(1))
        c = jnp.sum((s_ref[:, :] >= mid).astype(jnp.int32), axis=1,
                    keepdims=True)
        pred = c >= kk
        return jnp.where(pred, mid, lo), jnp.where(pred, hi, mid)

    thr_s, _ = jax.lax.fori_loop(0, 32, cnt_step, (lo0, hi0))

    mask1 = s_ref[:, :] >= thr_s
    e = e_ref[:, :]
    ssum = jnp.sum(jnp.where(mask1, e, 0.0), axis=1, keepdims=True)
    ps = pp * ssum

    def tail_step(_, carry):
        lo, hi = carry
        mid = lo + ((hi - lo) >> jnp.uint32(1))
        h = jnp.sum(jnp.where(s_ref[:, :] > mid, e_ref[:, :], 0.0), axis=1,
                    keepdims=True)
        pred = h >= ps
        return jnp.where(pred, mid, lo), jnp.where(pred, hi, mid)

    _, hi2 = jax.lax.fori_loop(0, 32, tail_step, (lo0, hi0))

    keep = mask1 & (s_ref[:, :] >= hi2)
    neg_inf = jnp.float32(-jnp.inf)
    out_ref[:, :] = jnp.where(keep, x, neg_inf)

    score = jnp.where(keep, x - g_ref[:, :], neg_inf)
    best = jnp.max(score, axis=1, keepdims=True)
    idx = jax.lax.broadcasted_iota(jnp.int32, score.shape, 1)
    samp_ref[:, :] = jnp.min(jnp.where(score == best, idx, jnp.int32(2**30)),
                             axis=1, keepdims=True)


def kernel(logits, k, p, gumbel):
    b, v = logits.shape
    vp = ((v + 127) // 128) * 128
    if vp != v:
        logits = jnp.pad(logits, ((0, 0), (0, vp - v)),
                         constant_values=_NEG_BIG)
        gumbel = jnp.pad(gumbel, ((0, 0), (0, vp - v)))
    k2 = k.reshape(b, 1).astype(jnp.int32)
    p2 = p.reshape(b, 1).astype(jnp.float32)

    r = _ROWS
    out, samp = pl.pallas_call(
        _block,
        grid=(b // r,),
        in_specs=[
            pl.BlockSpec((r, 1), lambda i: (i, 0)),
            pl.BlockSpec((r, 1), lambda i: (i, 0)),
            pl.BlockSpec((r, vp), lambda i: (i, 0)),
            pl.BlockSpec((r, vp), lambda i: (i, 0)),
        ],
        out_specs=[
            pl.BlockSpec((r, vp), lambda i: (i, 0)),
            pl.BlockSpec((r, 1), lambda i: (i, 0)),
        ],
        out_shape=[
            jax.ShapeDtypeStruct((b, vp), jnp.float32),
            jax.ShapeDtypeStruct((b, 1), jnp.int32),
        ],
        scratch_shapes=[
            pltpu.VMEM((r, vp), jnp.float32),
            pltpu.VMEM((r, vp), jnp.uint32),
        ],
        compiler_params=pltpu.CompilerParams(
            dimension_semantics=("parallel",)),
    )(k2, p2, logits, gumbel)
    return samp.reshape(-1), out[:, :v]


# speedup vs baseline: 56.0877x; 1.2460x over previous
import jax
import jax.numpy as jnp
from jax.experimental import pallas as pl
from jax.experimental.pallas import tpu as pltpu

_NEG_BIG = -3.0e38
_ROWS = 8


def _sortable(x):
    u = jax.lax.bitcast_convert_type(x, jnp.uint32)
    top = jnp.uint32(0x80000000)
    return jnp.where(u >= top, ~u, u | top)


def _block(k_ref, p_ref, x_ref, g_ref, out_ref, samp_ref, e_ref, s_ref):
    x = x_ref[:, :]
    kk = k_ref[:, :]
    pp = p_ref[:, :]
    vp = x.shape[1]

    m = jnp.max(x, axis=1, keepdims=True)
    s_ref[:, :] = _sortable(x)
    e_ref[:, :] = jnp.exp(x - m)

    lo0 = jnp.zeros(kk.shape, jnp.uint32)
    hi0 = jnp.full(kk.shape, jnp.uint32(0xFFFFFFFF))

    def done1(lo, hi, clo, chi):
        return ((clo - chi) == 1) | ((hi - lo) <= jnp.uint32(1))

    def cond1(carry):
        i, lo, hi, clo, chi = carry
        return jnp.logical_and(i < 34, ~jnp.all(done1(lo, hi, clo, chi)))

    def body1(carry):
        i, lo, hi, clo, chi = carry
        act = ~done1(lo, hi, clo, chi)
        mid = lo + ((hi - lo) >> jnp.uint32(1))
        c = jnp.sum((s_ref[:, :] >= mid).astype(jnp.int32), axis=1,
                    keepdims=True)
        up = act & (c >= kk)
        dn = act & (c < kk)
        return (i + 1, jnp.where(up, mid, lo), jnp.where(dn, mid, hi),
                jnp.where(up, c, clo), jnp.where(dn, c, chi))

    _, lo1, _, c1, _ = jax.lax.while_loop(
        cond1, body1,
        (jnp.int32(0), lo0, hi0, jnp.full(kk.shape, jnp.int32(vp)),
         jnp.zeros(kk.shape, jnp.int32)))

    mask1 = s_ref[:, :] >= lo1
    e = e_ref[:, :]
    ssum = jnp.sum(jnp.where(mask1, e, 0.0), axis=1, keepdims=True)
    ps = pp * ssum

    def done2(lo, hi, dlo, dhi):
        return ((dlo - dhi) == 1) | ((hi - lo) <= jnp.uint32(1))

    def cond2(carry):
        i, lo, hi, dlo, dhi = carry
        return jnp.logical_and(i < 34, ~jnp.all(done2(lo, hi, dlo, dhi)))

    def body2(carry):
        i, lo, hi, dlo, dhi = carry
        act = ~done2(lo, hi, dlo, dhi)
        mid = lo + ((hi - lo) >> jnp.uint32(1))
        gt = s_ref[:, :] > mid
        h = jnp.sum(jnp.where(gt, e_ref[:, :], 0.0), axis=1, keepdims=True)
        d = jnp.sum(gt.astype(jnp.int32), axis=1, keepdims=True)
        up = act & (h >= ps)
        dn = act & (h < ps)
        return (i + 1, jnp.where(up, mid, lo), jnp.where(dn, mid, hi),
                jnp.where(up, d, dlo), jnp.where(dn, d, dhi))

    _, lo2, _, _, _ = jax.lax.while_loop(
        cond2, body2,
        (jnp.int32(0), lo1 - jnp.uint32(1), _sortable(m) + jnp.uint32(1),
         c1, jnp.zeros(kk.shape, jnp.int32)))

    keep = mask1 & (s_ref[:, :] > lo2)
    neg_inf = jnp.float32(-jnp.inf)
    out_ref[:, :] = jnp.where(keep, x, neg_inf)

    score = jnp.where(keep, x - g_ref[:, :], neg_inf)
    best = jnp.max(score, axis=1, keepdims=True)
    idx = jax.lax.broadcasted_iota(jnp.int32, score.shape, 1)
    samp_ref[:, :] = jnp.min(jnp.where(score == best, idx, jnp.int32(2**30)),
                             axis=1, keepdims=True)


def kernel(logits, k, p, gumbel):
    b, v = logits.shape
    vp = ((v + 127) // 128) * 128
    if vp != v:
        logits = jnp.pad(logits, ((0, 0), (0, vp - v)),
                         constant_values=_NEG_BIG)
        gumbel = jnp.pad(gumbel, ((0, 0), (0, vp - v)))
    k2 = k.reshape(b, 1).astype(jnp.int32)
    p2 = p.reshape(b, 1).astype(jnp.float32)

    r = _ROWS
    out, samp = pl.pallas_call(
        _block,
        grid=(b // r,),
        in_specs=[
            pl.BlockSpec((r, 1), lambda i: (i, 0)),
            pl.BlockSpec((r, 1), lambda i: (i, 0)),
            pl.BlockSpec((r, vp), lambda i: (i, 0)),
            pl.BlockSpec((r, vp), lambda i: (i, 0)),
        ],
        out_specs=[
            pl.BlockSpec((r, vp), lambda i: (i, 0)),
            pl.BlockSpec((r, 1), lambda i: (i, 0)),
        ],
        out_shape=[
            jax.ShapeDtypeStruct((b, vp), jnp.float32),
            jax.ShapeDtypeStruct((b, 1), jnp.int32),
        ],
        scratch_shapes=[
            pltpu.VMEM((r, vp), jnp.float32),
            pltpu.VMEM((r, vp), jnp.uint32),
        ],
        compiler_params=pltpu.CompilerParams(
            dimension_semantics=("parallel",)),
    )(k2, p2, logits, gumbel)
    return samp.reshape(-1), out[:, :v]
